# baseline (speedup 1.0000x reference)
import functools

import jax
import jax.numpy as jnp
from jax import lax
from jax.experimental import pallas as pl
from jax.experimental.pallas import tpu as pltpu
from jax.experimental.pallas import tpu_sc as plsc

_NC = 2
_NS = 16
_NW = _NC * _NS
_D = 128
_B = 4 * 8192
_BP = _B // _NW
_C = 128
_NCH = _BP // _C
_RING = 3


@functools.partial(
    pl.kernel,
    mesh=plsc.VectorSubcoreMesh(core_axis_name="c", subcore_axis_name="s"),
    out_type=[
        jax.ShapeDtypeStruct((_B, _D), jnp.float32),
        jax.ShapeDtypeStruct((_B, _D), jnp.float32),
    ],
    scratch_types=(
        [pltpu.VMEM((_NCH, _C), jnp.int32)]
        + [pltpu.VMEM((_C, _D), jnp.float32) for _ in range(2 * _RING)]
        + [pltpu.SemaphoreType.DMA for _ in range(2 * _RING)]
    ),
)
def _rope_gather(pos_hbm, cos_hbm, sin_hbm, cos_out, sin_out, idx_v, *rest):
    cosb = rest[0:_RING]
    sinb = rest[_RING:2 * _RING]
    sg = rest[2 * _RING:3 * _RING]
    sw = rest[3 * _RING:4 * _RING]
    wid = lax.axis_index("s") * _NC + lax.axis_index("c")
    base = wid * _BP
    pltpu.sync_copy(pos_hbm.at[wid], idx_v)
    gh, wh = {}, {}
    for c in range(_NCH):
        b = c % _RING
        if c >= _RING:
            for h in wh[c - _RING]:
                h.wait()
        gh[c] = (pltpu.async_copy(cos_hbm.at[idx_v.at[c]], cosb[b], sg[b]),
                 pltpu.async_copy(sin_hbm.at[idx_v.at[c]], sinb[b], sg[b]))
        if c >= 1:
            p, pb = c - 1, (c - 1) % _RING
            for h in gh[p]:
                h.wait()
            off = base + p * _C
            wh[p] = (pltpu.async_copy(cosb[pb], cos_out.at[pl.ds(off, _C)], sw[pb]),
                     pltpu.async_copy(sinb[pb], sin_out.at[pl.ds(off, _C)], sw[pb]))
    last, lb = _NCH - 1, (_NCH - 1) % _RING
    for h in gh[last]:
        h.wait()
    off = base + last * _C
    wh[last] = (pltpu.async_copy(cosb[lb], cos_out.at[pl.ds(off, _C)], sw[lb]),
                pltpu.async_copy(sinb[lb], sin_out.at[pl.ds(off, _C)], sw[lb]))
    for c in range(max(0, _NCH - _RING), _NCH):
        for h in wh[c]:
            h.wait()


def kernel(positions, cos_cached, sin_cached):
    shape = positions.shape
    pos = positions.reshape(_NW, _NCH, _C)
    cos, sin = _rope_gather(pos, cos_cached, sin_cached)
    return (cos.reshape(*shape, _D), sin.reshape(*shape, _D))


# speedup vs baseline: 1.1960x; 1.1960x over previous
import functools

import jax
import jax.numpy as jnp
from jax import lax
from jax.experimental import pallas as pl
from jax.experimental.pallas import tpu as pltpu
from jax.experimental.pallas import tpu_sc as plsc

_NC = 2
_NS = 16
_NW = _NC * _NS
_D = 128
_DH = _D // 2
_B = 4 * 8192
_BP = _B // _NW
_C = 128
_NCH = _BP // _C
_RING = 3


@functools.partial(
    pl.kernel,
    mesh=plsc.VectorSubcoreMesh(core_axis_name="c", subcore_axis_name="s"),
    out_type=[
        jax.ShapeDtypeStruct((_B, _D), jnp.float32),
        jax.ShapeDtypeStruct((_B, _D), jnp.float32),
    ],
    scratch_types=(
        [pltpu.VMEM((_NCH, _C), jnp.int32)]
        + [pltpu.VMEM((_C, _DH), jnp.float32) for _ in range(2 * _RING)]
        + [pltpu.SemaphoreType.DMA for _ in range(2 * _RING)]
    ),
    compiler_params=pltpu.CompilerParams(use_tc_tiling_on_sc=False),
)
def _rope_gather(pos_hbm, cos_hbm, sin_hbm, cos_out, sin_out, idx_v, *rest):
    cosb = rest[0:_RING]
    sinb = rest[_RING:2 * _RING]
    sg = rest[2 * _RING:3 * _RING]
    sw = rest[3 * _RING:4 * _RING]
    wid = lax.axis_index("s") * _NC + lax.axis_index("c")
    base = wid * _BP
    pltpu.sync_copy(pos_hbm.at[wid], idx_v)
    gh, wh = {}, {}

    def issue_writes(c):
        b = c % _RING
        off = base + c * _C
        return tuple(
            pltpu.async_copy(buf, out.at[pl.ds(off, _C), pl.ds(col, _DH)], sw[b])
            for buf, out in ((cosb[b], cos_out), (sinb[b], sin_out))
            for col in (0, _DH)
        )

    for c in range(_NCH):
        b = c % _RING
        if c >= _RING:
            for h in wh[c - _RING]:
                h.wait()
        gh[c] = (pltpu.async_copy(cos_hbm.at[idx_v.at[c]], cosb[b], sg[b]),
                 pltpu.async_copy(sin_hbm.at[idx_v.at[c]], sinb[b], sg[b]))
        if c >= 1:
            for h in gh[c - 1]:
                h.wait()
            wh[c - 1] = issue_writes(c - 1)
    for h in gh[_NCH - 1]:
        h.wait()
    wh[_NCH - 1] = issue_writes(_NCH - 1)
    for c in range(max(0, _NCH - _RING), _NCH):
        for h in wh[c]:
            h.wait()


def kernel(positions, cos_cached, sin_cached):
    shape = positions.shape
    n_rows, d = cos_cached.shape
    pos2 = (positions * 2).reshape(_NW, _NCH, _C)
    cos_h = cos_cached.reshape(n_rows * 2, d // 2)
    sin_h = sin_cached.reshape(n_rows * 2, d // 2)
    cos, sin = _rope_gather(pos2, cos_h, sin_h)
    return (cos.reshape(*shape, _D), sin.reshape(*shape, _D))


# speedup vs baseline: 1.2309x; 1.0292x over previous
import functools

import jax
import jax.numpy as jnp
from jax import lax
from jax.experimental import pallas as pl
from jax.experimental.pallas import tpu as pltpu
from jax.experimental.pallas import tpu_sc as plsc

_NC = 2
_NS = 16
_NW = _NC * _NS
_D = 128
_DH = _D // 2
_B = 4 * 8192
_BP = _B // _NW
_C = 128
_NCH = _BP // _C
_SC = 2
_CS = _C * _SC
_NSC = _NCH // _SC
_RING = 3


@functools.partial(
    pl.kernel,
    mesh=plsc.VectorSubcoreMesh(core_axis_name="c", subcore_axis_name="s"),
    out_type=[
        jax.ShapeDtypeStruct((_B, _D), jnp.float32),
        jax.ShapeDtypeStruct((_B, _D), jnp.float32),
    ],
    scratch_types=(
        [pltpu.VMEM((_NCH, _C), jnp.int32)]
        + [pltpu.VMEM((_CS, _DH), jnp.float32) for _ in range(2 * _RING)]
        + [pltpu.SemaphoreType.DMA for _ in range(2 * _RING)]
    ),
    compiler_params=pltpu.CompilerParams(use_tc_tiling_on_sc=False),
)
def _rope_gather(pos_hbm, cos_hbm, sin_hbm, cos_out, sin_out, idx_v, *rest):
    cosb = rest[0:_RING]
    sinb = rest[_RING:2 * _RING]
    sg = rest[2 * _RING:3 * _RING]
    sw = rest[3 * _RING:4 * _RING]
    wid = lax.axis_index("s") * _NC + lax.axis_index("c")
    base = wid * _BP
    pltpu.sync_copy(pos_hbm.at[wid], idx_v)
    gh, wh = {}, {}

    def issue_gathers(s):
        b = s % _RING
        return tuple(
            pltpu.async_copy(tab.at[idx_v.at[s * _SC + j]],
                             buf.at[pl.ds(j * _C, _C)], sg[b])
            for tab, buf in ((cos_hbm, cosb[b]), (sin_hbm, sinb[b]))
            for j in range(_SC)
        )

    def issue_writes(s):
        b = s % _RING
        off = base + s * _CS
        return tuple(
            pltpu.async_copy(buf, out.at[pl.ds(off, _CS), pl.ds(col, _DH)], sw[b])
            for buf, out in ((cosb[b], cos_out), (sinb[b], sin_out))
            for col in (0, _DH)
        )

    for s in range(_NSC):
        if s >= _RING:
            for h in wh[s - _RING]:
                h.wait()
        gh[s] = issue_gathers(s)
        if s >= 1:
            for h in gh[s - 1]:
                h.wait()
            wh[s - 1] = issue_writes(s - 1)
    for h in gh[_NSC - 1]:
        h.wait()
    wh[_NSC - 1] = issue_writes(_NSC - 1)
    for s in range(max(0, _NSC - _RING), _NSC):
        for h in wh[s]:
            h.wait()


def kernel(positions, cos_cached, sin_cached):
    shape = positions.shape
    n_rows, d = cos_cached.shape
    pos2 = (positions * 2).reshape(_NW, _NCH, _C)
    cos_h = cos_cached.reshape(n_rows * 2, d // 2)
    sin_h = sin_cached.reshape(n_rows * 2, d // 2)
    cos, sin = _rope_gather(pos2, cos_h, sin_h)
    return (cos.reshape(*shape, _D), sin.reshape(*shape, _D))


# speedup vs baseline: 1.2336x; 1.0022x over previous
import functools

import jax
import jax.numpy as jnp
from jax import lax
from jax.experimental import pallas as pl
from jax.experimental.pallas import tpu as pltpu
from jax.experimental.pallas import tpu_sc as plsc

_NC = 2
_NS = 16
_NW = _NC * _NS
_D = 128
_DH = _D // 2
_B = 4 * 8192
_BP = _B // _NW
_C = 128
_NCH = _BP // _C
_SC = 2
_CS = _C * _SC
_NSC = _NCH // _SC
_RING = 3


@functools.partial(
    pl.kernel,
    mesh=plsc.VectorSubcoreMesh(core_axis_name="c", subcore_axis_name="s"),
    out_type=[
        jax.ShapeDtypeStruct((_B, _D), jnp.float32),
        jax.ShapeDtypeStruct((_B, _D), jnp.float32),
    ],
    scratch_types=(
        [pltpu.VMEM((_NCH, _C), jnp.int32)]
        + [pltpu.VMEM((_CS, _DH), jnp.float32) for _ in range(2 * _RING)]
        + [pltpu.SemaphoreType.DMA for _ in range(2 * _RING)]
    ),
    compiler_params=pltpu.CompilerParams(use_tc_tiling_on_sc=False),
)
def _rope_gather(pos_hbm, cos_hbm, sin_hbm, cos_out, sin_out, idx_v, *rest):
    cosb = rest[0:_RING]
    sinb = rest[_RING:2 * _RING]
    sg = rest[2 * _RING:3 * _RING]
    sw = rest[3 * _RING:4 * _RING]
    wid = lax.axis_index("s") * _NC + lax.axis_index("c")
    base = wid * _BP
    gh, wh = {}, {}

    def issue_gathers(s):
        b = s % _RING
        return tuple(
            pltpu.async_copy(tab.at[idx_v.at[s * _SC + j]],
                             buf.at[pl.ds(j * _C, _C)], sg[b])
            for tab, buf in ((cos_hbm, cosb[b]), (sin_hbm, sinb[b]))
            for j in range(_SC)
        )

    def issue_writes(s):
        b = s % _RING
        off = base + s * _CS
        return tuple(
            pltpu.async_copy(buf, out.at[pl.ds(off, _CS), pl.ds(col, _DH)], sw[b])
            for buf, out in ((cosb[b], cos_out), (sinb[b], sin_out))
            for col in (0, _DH)
        )

    pltpu.sync_copy(pos_hbm.at[wid, pl.ds(0, _SC)], idx_v.at[pl.ds(0, _SC)])
    gh[0] = issue_gathers(0)
    pltpu.sync_copy(pos_hbm.at[wid, pl.ds(_SC, _NCH - _SC)],
                    idx_v.at[pl.ds(_SC, _NCH - _SC)])
    if _NSC > 1:
        gh[1] = issue_gathers(1)
    waited_w = set()
    for s in range(_NSC):
        for h in gh[s]:
            h.wait()
        wh[s] = issue_writes(s)
        if s + 2 < _NSC:
            p = s + 2 - _RING
            if p >= 0:
                for h in wh[p]:
                    h.wait()
                waited_w.add(p)
            gh[s + 2] = issue_gathers(s + 2)
    for s in range(_NSC):
        if s not in waited_w:
            for h in wh[s]:
                h.wait()


def kernel(positions, cos_cached, sin_cached):
    shape = positions.shape
    n_rows, d = cos_cached.shape
    pos2 = (positions * 2).reshape(_NW, _NCH, _C)
    cos_h = cos_cached.reshape(n_rows * 2, d // 2)
    sin_h = sin_cached.reshape(n_rows * 2, d // 2)
    cos, sin = _rope_gather(pos2, cos_h, sin_h)
    return (cos.reshape(*shape, _D), sin.reshape(*shape, _D))
